# Initial kernel scaffold; baseline (speedup 1.0000x reference)
import jax
import jax.numpy as jnp
from jax.experimental import pallas as pl


def kernel(pooled_output, output_indices, W, b):
    raise NotImplementedError("write your pallas kernel here")


# speedup vs baseline: 1.9299x; 1.9299x over previous
import jax
import jax.numpy as jnp
from jax import lax
from jax.experimental import pallas as pl
from jax.experimental.pallas import tpu as pltpu
from jax.experimental.pallas import tpu_sc as plsc

NUM_LABELS = 100000
HIDDEN = 768
BATCH = 4096
SHORTLIST = 100

TOT = BATCH * SHORTLIST
NC, NS = 2, 16
NW = NC * NS
PER_W = TOT // NW
CHUNK = 64
NCHUNK = PER_W // CHUNK
NBUF = 2
NROWS = NUM_LABELS + 1
LANES = 16


def _w_body(w_hbm, idx_hbm, outw_hbm, idx_v, rows_v, sg0, sg1, so0, so1):
  sg = (sg0, sg1)
  so = (so0, so1)
  wid = lax.axis_index("s") * NC + lax.axis_index("c")
  base = wid * PER_W

  pltpu.sync_copy(idx_hbm.at[wid], idx_v)

  def out_slice(cid):
    return outw_hbm.at[pl.ds(base + cid * CHUNK, CHUNK)]

  @pl.loop(0, NCHUNK, step=NBUF)
  def _(j):
    for t in range(NBUF):
      cid = j + t

      @pl.when(cid >= NBUF)
      def _():
        pltpu.make_async_copy(rows_v.at[t], out_slice(cid - NBUF), so[t]).wait()

      pltpu.async_copy(w_hbm.at[idx_v.at[cid]], rows_v.at[t], sg[t])
    for t in range(NBUF):
      cid = j + t
      pltpu.make_async_copy(w_hbm.at[idx_v.at[cid]], rows_v.at[t], sg[t]).wait()
      pltpu.async_copy(rows_v.at[t], out_slice(cid), so[t])

  for t in range(NBUF):
    pltpu.make_async_copy(
        rows_v.at[t], out_slice(NCHUNK - NBUF + t), so[t]).wait()


def _b_body(b_hbm, idx_hbm, outb_hbm, tab_v, idx_v, out_v):
  wid = lax.axis_index("s") * NC + lax.axis_index("c")
  base = wid * PER_W

  pltpu.sync_copy(b_hbm, tab_v)
  pltpu.sync_copy(idx_hbm.at[pl.ds(base, PER_W)], idx_v)

  @pl.loop(0, PER_W // LANES)
  def _(i):
    iv = idx_v[pl.ds(i * LANES, LANES)]
    out_v[pl.ds(i * LANES, LANES)] = plsc.load_gather(tab_v, [iv])

  pltpu.sync_copy(out_v, outb_hbm.at[pl.ds(base, PER_W)])


@jax.jit
def _gather(w, idx3, idx_flat, b_flat):
  mesh = plsc.VectorSubcoreMesh(core_axis_name="c", subcore_axis_name="s")
  outw = pl.kernel(
      _w_body,
      out_type=jax.ShapeDtypeStruct((TOT, HIDDEN), jnp.float32),
      mesh=mesh,
      scratch_types=(
          pltpu.VMEM((NCHUNK, CHUNK), jnp.int32),
          pltpu.VMEM((NBUF, CHUNK, HIDDEN), jnp.float32),
          pltpu.SemaphoreType.DMA,
          pltpu.SemaphoreType.DMA,
          pltpu.SemaphoreType.DMA,
          pltpu.SemaphoreType.DMA,
      ),
  )(w, idx3)
  outb = pl.kernel(
      _b_body,
      out_type=jax.ShapeDtypeStruct((TOT,), jnp.float32),
      mesh=mesh,
      scratch_types=(
          pltpu.VMEM((NROWS,), jnp.float32),
          pltpu.VMEM((PER_W,), jnp.int32),
          pltpu.VMEM((PER_W,), jnp.float32),
      ),
      compiler_params=pltpu.CompilerParams(needs_layout_passes=False),
  )(b_flat, idx_flat)
  return outw, outb


def kernel(pooled_output, output_indices, W, b):
  del pooled_output
  idx_flat = output_indices.reshape(TOT)
  idx3 = idx_flat.reshape(NW, NCHUNK, CHUNK)
  outw, outb = _gather(W, idx3, idx_flat, b.reshape(NROWS))
  return (outw.reshape(BATCH, SHORTLIST, HIDDEN),
          outb.reshape(BATCH, SHORTLIST, 1))


# speedup vs baseline: 6.0894x; 3.1553x over previous
import jax
import jax.numpy as jnp
from jax import lax
from jax.experimental import pallas as pl
from jax.experimental.pallas import tpu as pltpu
from jax.experimental.pallas import tpu_sc as plsc

NUM_LABELS = 100000
HIDDEN = 768
BATCH = 4096
SHORTLIST = 100

TOT = BATCH * SHORTLIST
NC, NS = 2, 16
NW = NC * NS
PER_W = TOT // NW
CHUNK = 64
NCHUNK = PER_W // CHUNK
NBUF = 2
NROWS = NUM_LABELS + 1
LANES = 16


def _w_body(w_hbm, idx_hbm, outw_hbm, idx_v, rows_v, sg0, sg1, so0, so1):
  sg = (sg0, sg1)
  so = (so0, so1)
  wid = lax.axis_index("s") * NC + lax.axis_index("c")
  base = wid * PER_W

  pltpu.sync_copy(idx_hbm.at[wid], idx_v)

  def out_slice(cid):
    return outw_hbm.at[pl.ds(base + cid * CHUNK, CHUNK)]

  @pl.loop(0, NCHUNK, step=NBUF)
  def _(j):
    for t in range(NBUF):
      cid = j + t

      @pl.when(cid >= NBUF)
      def _():
        pltpu.make_async_copy(rows_v.at[t], out_slice(cid - NBUF), so[t]).wait()

      pltpu.async_copy(w_hbm.at[idx_v.at[cid]], rows_v.at[t], sg[t])
    for t in range(NBUF):
      cid = j + t
      pltpu.make_async_copy(w_hbm.at[idx_v.at[cid]], rows_v.at[t], sg[t]).wait()
      pltpu.async_copy(rows_v.at[t], out_slice(cid), so[t])

  for t in range(NBUF):
    pltpu.make_async_copy(
        rows_v.at[t], out_slice(NCHUNK - NBUF + t), so[t]).wait()


def _b_body(b_hbm, idx_hbm, outb_hbm, tab_v, idx_v, out_v):
  wid = lax.axis_index("s") * NC + lax.axis_index("c")
  base = wid * PER_W

  pltpu.sync_copy(b_hbm, tab_v)
  pltpu.sync_copy(idx_hbm.at[pl.ds(base, PER_W)], idx_v)

  @pl.loop(0, PER_W // LANES)
  def _(i):
    iv = idx_v[pl.ds(i * LANES, LANES)]
    out_v[pl.ds(i * LANES, LANES)] = plsc.load_gather(tab_v, [iv])

  pltpu.sync_copy(out_v, outb_hbm.at[pl.ds(base, PER_W)])


@jax.jit
def _gather(w, idx3, idx_flat, b_flat):
  mesh = plsc.VectorSubcoreMesh(core_axis_name="c", subcore_axis_name="s")
  outw = pl.kernel(
      _w_body,
      out_type=jax.ShapeDtypeStruct((TOT, HIDDEN), jnp.float32),
      mesh=mesh,
      scratch_types=(
          pltpu.VMEM((NCHUNK, CHUNK), jnp.int32),
          pltpu.VMEM((NBUF, CHUNK, HIDDEN), jnp.float32),
          pltpu.SemaphoreType.DMA,
          pltpu.SemaphoreType.DMA,
          pltpu.SemaphoreType.DMA,
          pltpu.SemaphoreType.DMA,
      ),
  )(w, idx3)
  outb = pl.kernel(
      _b_body,
      out_type=jax.ShapeDtypeStruct((TOT,), jnp.float32),
      mesh=mesh,
      scratch_types=(
          pltpu.VMEM((NROWS,), jnp.float32),
          pltpu.VMEM((PER_W,), jnp.int32),
          pltpu.VMEM((PER_W,), jnp.float32),
      ),
      compiler_params=pltpu.CompilerParams(needs_layout_passes=False),
  )(b_flat, idx_flat)
  return outw, outb


def kernel(pooled_output, output_indices, W, b):
  del pooled_output
  idx_flat = output_indices.T.reshape(TOT)
  idx3 = idx_flat.reshape(NW, NCHUNK, CHUNK)
  outw, outb = _gather(W, idx3, idx_flat, b.reshape(NROWS))
  w_act = outw.reshape(SHORTLIST, BATCH, HIDDEN).transpose(1, 0, 2)
  b_act = outb.reshape(SHORTLIST, BATCH, 1).transpose(1, 0, 2)
  return (w_act, b_act)


# speedup vs baseline: 6.0902x; 1.0001x over previous
import jax
import jax.numpy as jnp
from jax import lax
from jax.experimental import pallas as pl
from jax.experimental.pallas import tpu as pltpu
from jax.experimental.pallas import tpu_sc as plsc

NUM_LABELS = 100000
HIDDEN = 768
BATCH = 4096
SHORTLIST = 100

TOT = BATCH * SHORTLIST
NC, NS = 2, 16
NW = NC * NS
PER_W = TOT // NW
CHUNK = 32
NCHUNK = PER_W // CHUNK
NBUF = 4
NROWS = NUM_LABELS + 1
LANES = 16
def _w_body(w_hbm, idx_hbm, outw_hbm, idx_v, rows_v,
            sg0, sg1, sg2, sg3, so0, so1, so2, so3):
  sg = (sg0, sg1, sg2, sg3)
  so = (so0, so1, so2, so3)
  wid = lax.axis_index("s") * NC + lax.axis_index("c")
  base = wid * PER_W

  pltpu.sync_copy(idx_hbm.at[pl.ds(base, PER_W)], idx_v)

  def idx_slice(cid):
    return idx_v.at[pl.ds(cid * CHUNK, CHUNK)]

  def out_slice(cid):
    return outw_hbm.at[pl.ds(base + cid * CHUNK, CHUNK)]

  @pl.loop(0, NCHUNK, step=NBUF)
  def _(j):
    for t in range(NBUF):
      cid = j + t

      @pl.when(cid >= NBUF)
      def _():
        pltpu.make_async_copy(rows_v.at[t], out_slice(cid - NBUF), so[t]).wait()

      pltpu.async_copy(w_hbm.at[idx_slice(cid)], rows_v.at[t], sg[t])
    for t in range(NBUF):
      cid = j + t
      pltpu.make_async_copy(w_hbm.at[idx_slice(cid)], rows_v.at[t], sg[t]).wait()
      pltpu.async_copy(rows_v.at[t], out_slice(cid), so[t])

  for t in range(NBUF):
    pltpu.make_async_copy(
        rows_v.at[t], out_slice(NCHUNK - NBUF + t), so[t]).wait()


def _b_body(b_hbm, idx_hbm, outb_hbm, tab_v, idx_v, out_v):
  wid = lax.axis_index("s") * NC + lax.axis_index("c")
  base = wid * PER_W

  pltpu.sync_copy(b_hbm, tab_v)
  pltpu.sync_copy(idx_hbm.at[pl.ds(base, PER_W)], idx_v)

  @pl.loop(0, PER_W // LANES)
  def _(i):
    iv = idx_v[pl.ds(i * LANES, LANES)]
    out_v[pl.ds(i * LANES, LANES)] = plsc.load_gather(tab_v, [iv])

  pltpu.sync_copy(out_v, outb_hbm.at[pl.ds(base, PER_W)])


@jax.jit
def _gather(w, idx_flat, b_flat):
  mesh = plsc.VectorSubcoreMesh(core_axis_name="c", subcore_axis_name="s")
  outw = pl.kernel(
      _w_body,
      out_type=jax.ShapeDtypeStruct((TOT, HIDDEN), jnp.float32),
      mesh=mesh,
      scratch_types=(
          pltpu.VMEM((PER_W,), jnp.int32),
          pltpu.VMEM((NBUF, CHUNK, HIDDEN), jnp.float32),
          pltpu.SemaphoreType.DMA,
          pltpu.SemaphoreType.DMA,
          pltpu.SemaphoreType.DMA,
          pltpu.SemaphoreType.DMA,
          pltpu.SemaphoreType.DMA,
          pltpu.SemaphoreType.DMA,
          pltpu.SemaphoreType.DMA,
          pltpu.SemaphoreType.DMA,
      ),
  )(w, idx_flat)
  outb = pl.kernel(
      _b_body,
      out_type=jax.ShapeDtypeStruct((TOT,), jnp.float32),
      mesh=mesh,
      scratch_types=(
          pltpu.VMEM((NROWS,), jnp.float32),
          pltpu.VMEM((PER_W,), jnp.int32),
          pltpu.VMEM((PER_W,), jnp.float32),
      ),
      compiler_params=pltpu.CompilerParams(needs_layout_passes=False),
  )(b_flat, idx_flat)
  return outw, outb


def kernel(pooled_output, output_indices, W, b):
  del pooled_output
  idx_flat = output_indices.T.reshape(TOT)
  outw, outb = _gather(W, idx_flat, b.reshape(NROWS))
  w_act = outw.reshape(SHORTLIST, BATCH, HIDDEN).transpose(1, 0, 2)
  b_act = outb.reshape(SHORTLIST, BATCH, 1).transpose(1, 0, 2)
  return (w_act, b_act)


# speedup vs baseline: 6.1640x; 1.0121x over previous
import jax
import jax.numpy as jnp
from jax import lax
from jax.experimental import pallas as pl
from jax.experimental.pallas import tpu as pltpu
from jax.experimental.pallas import tpu_sc as plsc

NUM_LABELS = 100000
HIDDEN = 768
BATCH = 4096
SHORTLIST = 100

TOT = BATCH * SHORTLIST
NC, NS = 2, 16
NW = NC * NS
PER_W = TOT // NW
CHUNK = 32
NCHUNK = PER_W // CHUNK
NBUF = 4
NROWS = NUM_LABELS + 1
LANES = 16


def _body(w_hbm, idx_hbm, b_hbm, outw_hbm, outb_hbm, idx_v,
          sg0, sg1, sg2, sg3, so0, so1, so2, so3):
  sg = (sg0, sg1, sg2, sg3)
  so = (so0, so1, so2, so3)
  wid = lax.axis_index("s") * NC + lax.axis_index("c")
  base = wid * PER_W

  pltpu.sync_copy(idx_hbm.at[pl.ds(base, PER_W)], idx_v)

  def idx_slice(cid):
    return idx_v.at[pl.ds(cid * CHUNK, CHUNK)]

  def out_slice(cid):
    return outw_hbm.at[pl.ds(base + cid * CHUNK, CHUNK)]

  def w_phase(rows_v):
    @pl.loop(0, NCHUNK, step=NBUF)
    def _(j):
      for t in range(NBUF):
        cid = j + t

        @pl.when(cid >= NBUF)
        def _():
          pltpu.make_async_copy(
              rows_v.at[t], out_slice(cid - NBUF), so[t]).wait()

        pltpu.async_copy(w_hbm.at[idx_slice(cid)], rows_v.at[t], sg[t])
      for t in range(NBUF):
        cid = j + t
        pltpu.make_async_copy(
            w_hbm.at[idx_slice(cid)], rows_v.at[t], sg[t]).wait()
        pltpu.async_copy(rows_v.at[t], out_slice(cid), so[t])

    for t in range(NBUF):
      pltpu.make_async_copy(
          rows_v.at[t], out_slice(NCHUNK - NBUF + t), so[t]).wait()

  pl.run_scoped(w_phase, pltpu.VMEM((NBUF, CHUNK, HIDDEN), jnp.float32))

  def b_phase(tab_v, out_v):
    pltpu.sync_copy(b_hbm, tab_v)

    @pl.loop(0, PER_W // LANES)
    def _(i):
      iv = idx_v[pl.ds(i * LANES, LANES)]
      out_v[pl.ds(i * LANES, LANES)] = plsc.load_gather(tab_v, [iv])

    pltpu.sync_copy(out_v, outb_hbm.at[pl.ds(base, PER_W)])

  pl.run_scoped(
      b_phase,
      pltpu.VMEM((NROWS,), jnp.float32),
      pltpu.VMEM((PER_W,), jnp.float32),
  )


@jax.jit
def _gather(w, idx_flat, b_flat):
  mesh = plsc.VectorSubcoreMesh(core_axis_name="c", subcore_axis_name="s")
  return pl.kernel(
      _body,
      out_type=(
          jax.ShapeDtypeStruct((TOT, HIDDEN), jnp.float32),
          jax.ShapeDtypeStruct((TOT,), jnp.float32),
      ),
      mesh=mesh,
      scratch_types=(
          pltpu.VMEM((PER_W,), jnp.int32),
          pltpu.SemaphoreType.DMA,
          pltpu.SemaphoreType.DMA,
          pltpu.SemaphoreType.DMA,
          pltpu.SemaphoreType.DMA,
          pltpu.SemaphoreType.DMA,
          pltpu.SemaphoreType.DMA,
          pltpu.SemaphoreType.DMA,
          pltpu.SemaphoreType.DMA,
      ),
      compiler_params=pltpu.CompilerParams(needs_layout_passes=False),
  )(w, idx_flat, b_flat)


def kernel(pooled_output, output_indices, W, b):
  del pooled_output
  idx_flat = output_indices.T.reshape(TOT)
  outw, outb = _gather(W, idx_flat, b.reshape(NROWS))
  w_act = outw.reshape(SHORTLIST, BATCH, HIDDEN).transpose(1, 0, 2)
  b_act = outb.reshape(SHORTLIST, BATCH, 1).transpose(1, 0, 2)
  return (w_act, b_act)
